# baseline (speedup 1.0000x reference)
import functools

import jax
import jax.numpy as jnp
from jax import lax
from jax.experimental import pallas as pl
from jax.experimental.pallas import tpu as pltpu
from jax.experimental.pallas import tpu_sc as plsc

_B = 4096
_L = 50
_DIM = 64
_N = _B * _L

_NC = 2
_NS = 16
_NW = _NC * _NS
_PER_W = _N // _NW

_CHUNK = 128
_SUB = 5
_GROUP = _CHUNK * _SUB
_STEPS = _PER_W // _GROUP
_ROWS_PER_W_2D = _PER_W // _CHUNK


def _sc_gather_body(tok_hbm, table_hbm, out_hbm, idx_v, rows_v, sem):
    wid = lax.axis_index("s") * _NC + lax.axis_index("c")
    flat0 = wid * _PER_W
    pltpu.sync_copy(tok_hbm.at[wid], idx_v)

    def step(g, carry):
        copies = []
        for j in range(_SUB):
            copies.append(
                pltpu.async_copy(
                    table_hbm.at[idx_v.at[g * _SUB + j]],
                    rows_v.at[pl.ds(j * _CHUNK, _CHUNK)],
                    sem,
                )
            )
        for cp in copies:
            cp.wait()
        pltpu.sync_copy(rows_v, out_hbm.at[pl.ds(flat0 + g * _GROUP, _GROUP)])
        return carry

    lax.fori_loop(0, _STEPS, step, 0)


_sc_gather = functools.partial(
    pl.kernel,
    out_type=jax.ShapeDtypeStruct((_N, _DIM), jnp.float32),
    mesh=plsc.VectorSubcoreMesh(core_axis_name="c", subcore_axis_name="s"),
    scratch_types=[
        pltpu.VMEM((_ROWS_PER_W_2D, _CHUNK), jnp.int32),
        pltpu.VMEM((_GROUP, _DIM), jnp.float32),
        pltpu.SemaphoreType.DMA,
    ],
    compiler_params=pltpu.CompilerParams(use_tc_tiling_on_sc=False),
)(_sc_gather_body)


_TC_BLK = 2048


def _tc_proj_body(x_ref, m_ref, w_ref, b_ref, o_ref):
    acc = lax.dot_general(
        x_ref[...], w_ref[...],
        (((1,), (1,)), ((), ())),
        preferred_element_type=jnp.float32,
    )
    o_ref[...] = (acc + b_ref[...]) * m_ref[...]


_tc_proj = pl.pallas_call(
    _tc_proj_body,
    grid=(_N // _TC_BLK,),
    in_specs=[
        pl.BlockSpec((_TC_BLK, _DIM), lambda i: (i, 0)),
        pl.BlockSpec((_TC_BLK, 1), lambda i: (i, 0)),
        pl.BlockSpec((_DIM, _DIM), lambda i: (0, 0)),
        pl.BlockSpec((1, _DIM), lambda i: (0, 0)),
    ],
    out_specs=pl.BlockSpec((_TC_BLK, _DIM), lambda i: (i, 0)),
    out_shape=jax.ShapeDtypeStruct((_N, _DIM), jnp.float32),
)


def kernel(tokens, mask, embed_table, W, b):
    tok3d = tokens.reshape(_NW, _ROWS_PER_W_2D, _CHUNK).astype(jnp.int32)
    gathered = _sc_gather(tok3d, embed_table)
    mf = mask.reshape(_N, 1).astype(jnp.float32)
    out = _tc_proj(gathered, mf, W, b.reshape(1, _DIM))
    return out.reshape(_B, _L, _DIM), mask


# speedup vs baseline: 1.1514x; 1.1514x over previous
import functools

import jax
import jax.numpy as jnp
from jax import lax
from jax.experimental import pallas as pl
from jax.experimental.pallas import tpu as pltpu
from jax.experimental.pallas import tpu_sc as plsc

_B = 4096
_L = 50
_DIM = 64
_N = _B * _L
_V = 1000000

_TBLK = 1024
_NBLK = (_V + _TBLK - 1) // _TBLK
_VP = _NBLK * (_TBLK // 2)

_NW = 32
_BW = _B // _NW


def _tc_prep_body(x_ref, w_ref, b_ref, o_ref):
    x = x_ref[...]
    cdims = (((0,), (1,)), ((), ()))
    a = lax.dot_general(x[:, :_TBLK // 2], w_ref[...], cdims,
                        preferred_element_type=jnp.float32)
    c = lax.dot_general(x[:, _TBLK // 2:], w_ref[...], cdims,
                        preferred_element_type=jnp.float32)
    bb = b_ref[...]
    o_ref[...] = jnp.concatenate([a + bb, c + bb], axis=1)


_tc_prep = pl.pallas_call(
    _tc_prep_body,
    grid=(_NBLK,),
    in_specs=[
        pl.BlockSpec((_DIM, _TBLK), lambda i: (0, i)),
        pl.BlockSpec((_DIM, _DIM), lambda i: (0, 0)),
        pl.BlockSpec((1, _DIM), lambda i: (0, 0)),
    ],
    out_specs=pl.BlockSpec((_TBLK // 2, 2 * _DIM), lambda i: (i, 0)),
    out_shape=jax.ShapeDtypeStruct((_VP, 2 * _DIM), jnp.float32),
)


def _sc_gather_body(idx_hbm, table_hbm, out_hbm, idx_v, rows_a, rows_b, sem_a, sem_b):
    wid = lax.axis_index("s") * 2 + lax.axis_index("c")
    col0 = wid * _BW
    pltpu.sync_copy(idx_hbm.at[:, pl.ds(col0, _BW)], idx_v)

    bufs = (rows_a, rows_b)
    sems = (sem_a, sem_b)

    pltpu.make_async_copy(table_hbm.at[idx_v.at[0]], rows_a, sem_a).start()

    def step(k, carry):
        for par in (0, 1):
            l = 2 * k + par
            buf = bufs[par]
            sem = sems[par]
            pltpu.make_async_copy(table_hbm.at[idx_v.at[l]], buf, sem).wait()
            nxt = l + 1

            @pl.when(nxt < _L)
            def _():
                pltpu.make_async_copy(
                    table_hbm.at[idx_v.at[nxt]], bufs[1 - par], sems[1 - par]
                ).start()

            pltpu.sync_copy(buf, out_hbm.at[pl.ds(l * _B + col0, _BW)])
        return carry

    lax.fori_loop(0, _L // 2, step, 0)


_sc_gather = functools.partial(
    pl.kernel,
    out_type=jax.ShapeDtypeStruct((_N, 2 * _DIM), jnp.float32),
    mesh=plsc.VectorSubcoreMesh(core_axis_name="c", subcore_axis_name="s"),
    scratch_types=[
        pltpu.VMEM((_L, _BW), jnp.int32),
        pltpu.VMEM((_BW, 2 * _DIM), jnp.float32),
        pltpu.VMEM((_BW, 2 * _DIM), jnp.float32),
        pltpu.SemaphoreType.DMA,
        pltpu.SemaphoreType.DMA,
    ],
    compiler_params=pltpu.CompilerParams(use_tc_tiling_on_sc=True),
)(_sc_gather_body)


def _tc_sel_body(x_ref, par_ref, m_ref, il_ref, ir_ref, o_ref):
    x = x_ref[...]
    cdims = (((1,), (1,)), ((), ()))
    a = lax.dot_general(il_ref[...], x, cdims,
                        preferred_element_type=jnp.float32)
    c = lax.dot_general(ir_ref[...], x, cdims,
                        preferred_element_type=jnp.float32)
    par = par_ref[0] != 0
    sel = jnp.where(par, c, a)
    o_ref[...] = sel * m_ref[0].astype(jnp.float32)


_tc_sel = pl.pallas_call(
    _tc_sel_body,
    grid=(_L,),
    in_specs=[
        pl.BlockSpec((_B, 2 * _DIM), lambda l: (l, 0)),
        pl.BlockSpec((1, 1, _B), lambda l: (l, 0, 0)),
        pl.BlockSpec((1, 1, _B), lambda l: (l, 0, 0)),
        pl.BlockSpec((_DIM, 2 * _DIM), lambda l: (0, 0)),
        pl.BlockSpec((_DIM, 2 * _DIM), lambda l: (0, 0)),
    ],
    out_specs=pl.BlockSpec((_DIM, _B), lambda l: (l, 0)),
    out_shape=jax.ShapeDtypeStruct((_L * _DIM, _B), jnp.float32),
)


def kernel(tokens, mask, embed_table, W, b):
    tok = tokens.astype(jnp.int32)
    idxT = (((tok >> 10) << 9) | (tok & 511)).T
    parT = ((tok >> 9) & 1).T.reshape(_L, 1, _B)
    maskT = mask.T.reshape(_L, 1, _B)
    tableT = embed_table.T
    p_tab = _tc_prep(tableT, W, b.reshape(1, _DIM))
    gathered = _sc_gather(idxT, p_tab)
    eye = jnp.eye(_DIM, dtype=jnp.float32)
    zero = jnp.zeros((_DIM, _DIM), jnp.float32)
    il = jnp.concatenate([eye, zero], axis=1)
    ir = jnp.concatenate([zero, eye], axis=1)
    outp = _tc_sel(gathered, parT, maskT, il, ir)
    out = outp.reshape(_L, _DIM, _B).transpose(2, 0, 1)
    return out, mask


# speedup vs baseline: 2.5237x; 2.1918x over previous
import functools

import jax
import jax.numpy as jnp
from jax import lax
from jax.experimental import pallas as pl
from jax.experimental.pallas import tpu as pltpu
from jax.experimental.pallas import tpu_sc as plsc

_B = 4096
_L = 50
_DIM = 64
_N = _B * _L
_V = 1000000

_TBLK = 16384
_NBLK = (_V + _TBLK - 1) // _TBLK
_VP = _NBLK * (_TBLK // 2)

_NW = 32
_BW = _B // _NW


def _tc_prep_body(x_ref, w_ref, b_ref, o_ref):
    x = x_ref[...]
    cdims = (((0,), (1,)), ((), ()))
    a = lax.dot_general(x[:, :_TBLK // 2], w_ref[...], cdims,
                        preferred_element_type=jnp.float32)
    c = lax.dot_general(x[:, _TBLK // 2:], w_ref[...], cdims,
                        preferred_element_type=jnp.float32)
    bb = b_ref[...]
    o_ref[...] = jnp.concatenate([a + bb, c + bb], axis=1)


_tc_prep = pl.pallas_call(
    _tc_prep_body,
    grid=(_NBLK,),
    in_specs=[
        pl.BlockSpec((_DIM, _TBLK), lambda i: (0, i)),
        pl.BlockSpec((_DIM, _DIM), lambda i: (0, 0)),
        pl.BlockSpec((1, _DIM), lambda i: (0, 0)),
    ],
    out_specs=pl.BlockSpec((_TBLK // 2, 2 * _DIM), lambda i: (i, 0)),
    out_shape=jax.ShapeDtypeStruct((_VP, 2 * _DIM), jnp.float32),
)


def _sc_gather_body(idx_hbm, table_hbm, out_hbm, idx_v, rows_a, rows_b, sem_a, sem_b):
    wid = lax.axis_index("s") * 2 + lax.axis_index("c")
    col0 = wid * _BW
    pltpu.sync_copy(idx_hbm.at[:, pl.ds(col0, _BW)], idx_v)

    bufs = (rows_a, rows_b)
    sems = (sem_a, sem_b)

    pltpu.make_async_copy(table_hbm.at[idx_v.at[0]], rows_a, sem_a).start()

    def step(k, carry):
        for par in (0, 1):
            l = 2 * k + par
            buf = bufs[par]
            sem = sems[par]
            pltpu.make_async_copy(table_hbm.at[idx_v.at[l]], buf, sem).wait()
            nxt = l + 1

            @pl.when(nxt < _L)
            def _():
                pltpu.make_async_copy(
                    table_hbm.at[idx_v.at[nxt]], bufs[1 - par], sems[1 - par]
                ).start()

            pltpu.sync_copy(buf, out_hbm.at[pl.ds(l * _B + col0, _BW)])
        return carry

    lax.fori_loop(0, _L // 2, step, 0)


_sc_gather = functools.partial(
    pl.kernel,
    out_type=jax.ShapeDtypeStruct((_N, 2 * _DIM), jnp.float32),
    mesh=plsc.VectorSubcoreMesh(core_axis_name="c", subcore_axis_name="s"),
    scratch_types=[
        pltpu.VMEM((_L, _BW), jnp.int32),
        pltpu.VMEM((_BW, 2 * _DIM), jnp.float32),
        pltpu.VMEM((_BW, 2 * _DIM), jnp.float32),
        pltpu.SemaphoreType.DMA,
        pltpu.SemaphoreType.DMA,
    ],
    compiler_params=pltpu.CompilerParams(use_tc_tiling_on_sc=True),
)(_sc_gather_body)


def _tc_sel_body(x_ref, par_ref, m_ref, il_ref, ir_ref, o_ref):
    x = x_ref[...]
    cdims = (((1,), (1,)), ((), ()))
    a = lax.dot_general(il_ref[...], x, cdims,
                        preferred_element_type=jnp.float32)
    c = lax.dot_general(ir_ref[...], x, cdims,
                        preferred_element_type=jnp.float32)
    par = par_ref[0] != 0
    sel = jnp.where(par, c, a)
    o_ref[...] = sel * m_ref[0].astype(jnp.float32)


_tc_sel = pl.pallas_call(
    _tc_sel_body,
    grid=(_L,),
    in_specs=[
        pl.BlockSpec((_B, 2 * _DIM), lambda l: (l, 0)),
        pl.BlockSpec((1, 1, _B), lambda l: (l, 0, 0)),
        pl.BlockSpec((1, 1, _B), lambda l: (l, 0, 0)),
        pl.BlockSpec((_DIM, 2 * _DIM), lambda l: (0, 0)),
        pl.BlockSpec((_DIM, 2 * _DIM), lambda l: (0, 0)),
    ],
    out_specs=pl.BlockSpec((_DIM, _B), lambda l: (l, 0)),
    out_shape=jax.ShapeDtypeStruct((_L * _DIM, _B), jnp.float32),
)


def kernel(tokens, mask, embed_table, W, b):
    tok = tokens.astype(jnp.int32)
    idxT = (((tok >> 14) << 13) | (tok & 8191)).T
    parT = ((tok >> 13) & 1).T.reshape(_L, 1, _B)
    maskT = mask.T.reshape(_L, 1, _B)
    tableT = embed_table.T
    p_tab = _tc_prep(tableT, W, b.reshape(1, _DIM))
    gathered = _sc_gather(idxT, p_tab)
    eye = jnp.eye(_DIM, dtype=jnp.float32)
    zero = jnp.zeros((_DIM, _DIM), jnp.float32)
    il = jnp.concatenate([eye, zero], axis=1)
    ir = jnp.concatenate([zero, eye], axis=1)
    outp = _tc_sel(gathered, parT, maskT, il, ir)
    out = outp.reshape(_L, _DIM, _B).transpose(2, 0, 1)
    return out, mask
